# baseline (speedup 1.0000x reference)
import jax
import jax.numpy as jnp
from jax import lax
from jax.experimental import pallas as pl
from jax.experimental.pallas import tpu as pltpu
from jax.experimental.pallas import tpu_sc as plsc

LUT_SIZE = 1000000
KEY_DIM = 32
DIGESTS = 2
HASH_C = 73244475

NC = 2
NS = 16
NW = NC * NS
LANES = 16

BATCH = 4096
HIST = 200
BB = 128
N_BLOCKS = HIST * (BATCH // BB)
BLK_W = N_BLOCKS // NW
IDS_W = BLK_W * BB


def _wrap64_py(v):
    v &= (1 << 64) - 1
    if v >= (1 << 63):
        v -= 1 << 64
    return v


def _salt32(salt: int) -> int:
    s = int(salt)
    s = _wrap64_py((s >> 16 ^ s) * HASH_C)
    s = _wrap64_py((s >> 16 ^ s) * HASH_C)
    sv = s >> 16 ^ s
    sv &= (1 << 32) - 1
    if sv >= (1 << 31):
        sv -= 1 << 32
    return sv


SALTS = tuple(_salt32(n) for n in range(DIGESTS))


def _hash_mod(xv, salt):
    c = jnp.int32(HASH_C)
    k = xv ^ jnp.int32(salt)
    k = (k >> 16 ^ k) * c
    k = (k >> 16 ^ k) * c
    k = k >> 16 ^ k
    return k % jnp.int32(LUT_SIZE)


def _make_kernel():
    mesh = plsc.VectorSubcoreMesh(
        core_axis_name="c", subcore_axis_name="s",
        num_cores=NC, num_subcores=NS)

    def body(x_hbm, lut_hbm, out_hbm, x_v, idx_v, rows_v, tp_v,
             sem_g0, sem_g1, sem_s0, sem_s1):
        sem_g = (sem_g0, sem_g1)
        sem_s = (sem_s0, sem_s1)
        wid = lax.axis_index("s") * NC + lax.axis_index("c")
        lane = lax.iota(jnp.int32, 16)
        lane2 = lane * 2

        pltpu.sync_copy(x_hbm.at[pl.ds(wid * IDS_W, IDS_W)], x_v)

        def prep(k, q):
            def hblk(i, carry):
                xv = x_v[pl.ds(k * BB + i * LANES, LANES)]
                for n in range(DIGESTS):
                    p = (i * LANES + lane) * DIGESTS + n
                    plsc.store_scatter(
                        idx_v.at[q], [p >> 7, p & 127],
                        _hash_mod(xv, SALTS[n]))
                return carry

            lax.fori_loop(0, BB // LANES, hblk, 0)
            for j in range(DIGESTS):
                pltpu.async_copy(
                    lut_hbm.at[idx_v.at[q, j]],
                    rows_v.at[q, pl.ds(j * BB, BB)], sem_g[q])

        def drain_g(q):
            for j in range(DIGESTS):
                pltpu.make_async_copy(
                    lut_hbm.at[pl.ds(0, BB)],
                    rows_v.at[q, pl.ds(j * BB, BB)], sem_g[q]).wait()

        def transpose(q):
            def tblk(i0, carry):
                rbase = i0 * (2 * LANES) + lane2
                for dd in range(KEY_DIM):
                    col = jnp.full((LANES,), dd, jnp.int32)
                    tp_v[q, dd, pl.ds(i0 * LANES, LANES)] = (
                        plsc.load_gather(rows_v.at[q], [rbase, col]))
                    tp_v[q, KEY_DIM + dd, pl.ds(i0 * LANES, LANES)] = (
                        plsc.load_gather(rows_v.at[q], [rbase + 1, col]))
                return carry

            lax.fori_loop(0, BB // LANES, tblk, 0)

        def store(k, q):
            jblk = wid * BLK_W + k
            h = (jblk // 256) * 8 + jblk % 8
            b1 = (jblk // 8) % 32
            for d1 in range(8):
                pltpu.async_copy(
                    tp_v.at[q, pl.ds(d1 * 8, 8)],
                    out_hbm.at[h, d1, b1], sem_s[q])

        def drain_s(q):
            for d1 in range(8):
                pltpu.make_async_copy(
                    out_hbm.at[0, d1, 0], tp_v.at[q, pl.ds(d1 * 8, 8)],
                    sem_s[q]).wait()

        prep(0, 0)

        def step(g, carry):
            for p in range(2):
                k = 2 * g + p

                @pl.when(k + 1 < BLK_W)
                def _():
                    prep(k + 1, 1 - p)

                drain_g(p)

                @pl.when(k >= 2)
                def _():
                    drain_s(p)

                transpose(p)
                store(k, p)
            return carry

        lax.fori_loop(0, BLK_W // 2, step, 0)
        for p in range(2):
            drain_s(p)

    return pl.kernel(
        body,
        out_type=jax.ShapeDtypeStruct((HIST, 8, BATCH // BB, 8, BB),
                                      jnp.float32),
        mesh=mesh,
        compiler_params=pltpu.CompilerParams(use_tc_tiling_on_sc=False,
                                             needs_layout_passes=False),
        scratch_types=[
            pltpu.VMEM((IDS_W,), jnp.int32),
            pltpu.VMEM((2, DIGESTS, BB), jnp.int32),
            pltpu.VMEM((2, DIGESTS * BB, KEY_DIM), jnp.float32),
            pltpu.VMEM((2, DIGESTS * KEY_DIM, BB), jnp.float32),
            pltpu.SemaphoreType.DMA,
            pltpu.SemaphoreType.DMA,
            pltpu.SemaphoreType.DMA,
            pltpu.SemaphoreType.DMA,
        ],
    )


def kernel(x, lut):
    xt = x.reshape(BATCH // BB, BB, HIST // 8, 8)
    xt = xt.transpose(2, 0, 3, 1).reshape(BATCH * HIST)
    out5 = _make_kernel()(xt, lut)
    return out5.transpose(2, 4, 0, 1, 3).reshape(BATCH, HIST,
                                                 DIGESTS * KEY_DIM)


# speedup vs baseline: 1.7945x; 1.7945x over previous
import jax
import jax.numpy as jnp
from jax import lax
from jax.experimental import pallas as pl
from jax.experimental.pallas import tpu as pltpu
from jax.experimental.pallas import tpu_sc as plsc

LUT_SIZE = 1000000
KEY_DIM = 32
DIGESTS = 2
HASH_C = 73244475

NC = 2
NS = 16
NW = NC * NS
LANES = 16

BATCH = 4096
HIST = 200
BB = 128
N_BLOCKS = HIST * (BATCH // BB)
BLK_W = N_BLOCKS // NW
IDS_W = BLK_W * BB


def _wrap64_py(v):
    v &= (1 << 64) - 1
    if v >= (1 << 63):
        v -= 1 << 64
    return v


def _salt32(salt: int) -> int:
    s = int(salt)
    s = _wrap64_py((s >> 16 ^ s) * HASH_C)
    s = _wrap64_py((s >> 16 ^ s) * HASH_C)
    sv = s >> 16 ^ s
    sv &= (1 << 32) - 1
    if sv >= (1 << 31):
        sv -= 1 << 32
    return sv


SALTS = tuple(_salt32(n) for n in range(DIGESTS))


def _hash_mod(xv, salt):
    c = jnp.int32(HASH_C)
    k = xv ^ jnp.int32(salt)
    k = (k >> 16 ^ k) * c
    k = (k >> 16 ^ k) * c
    k = k >> 16 ^ k
    return k % jnp.int32(LUT_SIZE)


def _make_kernel():
    mesh = plsc.VectorSubcoreMesh(
        core_axis_name="c", subcore_axis_name="s",
        num_cores=NC, num_subcores=NS)

    def body(x_hbm, lut_hbm, out_hbm, x_v, idx_v, rows_v, tp_v,
             sem_g0, sem_g1, sem_s0, sem_s1):
        sem_g = (sem_g0, sem_g1)
        sem_s = (sem_s0, sem_s1)
        wid = lax.axis_index("s") * NC + lax.axis_index("c")
        lane = lax.iota(jnp.int32, 16)
        lane2 = lane * 2

        pltpu.sync_copy(x_hbm.at[pl.ds(wid * IDS_W, IDS_W)], x_v)

        def prep(k, q):
            def hblk(i, carry):
                xv = x_v[pl.ds(k * BB + i * LANES, LANES)]
                for n in range(DIGESTS):
                    p = (i * LANES + lane) * DIGESTS + n
                    plsc.store_scatter(
                        idx_v.at[q], [p >> 7, p & 127],
                        _hash_mod(xv, SALTS[n]))
                return carry

            lax.fori_loop(0, BB // LANES, hblk, 0)
            for j in range(DIGESTS):
                pltpu.async_copy(
                    lut_hbm.at[idx_v.at[q, j]],
                    rows_v.at[q, pl.ds(j * BB, BB)], sem_g[q])

        def drain_g(q):
            for j in range(DIGESTS):
                pltpu.make_async_copy(
                    lut_hbm.at[pl.ds(0, BB)],
                    rows_v.at[q, pl.ds(j * BB, BB)], sem_g[q]).wait()

        def transpose(q):
            def tblk(i, carry):
                si = lane * 0 + i
                for n in range(DIGESTS):
                    for c0 in (0, LANES):
                        dvec = lane + (n * KEY_DIM + c0)
                        v = rows_v[q, 2 * i + n, pl.ds(c0, LANES)]
                        plsc.store_scatter(tp_v.at[q], [dvec, si], v)
                return carry

            lax.fori_loop(0, BB, tblk, 0)

        def store(k, q):
            jblk = wid * BLK_W + k
            h = (jblk // 256) * 8 + jblk % 8
            b1 = (jblk // 8) % 32
            for d1 in range(8):
                pltpu.async_copy(
                    tp_v.at[q, pl.ds(d1 * 8, 8), pl.ds(0, BB)],
                    out_hbm.at[h, d1, b1], sem_s[q])

        def drain_s(q):
            for d1 in range(8):
                pltpu.make_async_copy(
                    out_hbm.at[0, d1, 0],
                    tp_v.at[q, pl.ds(d1 * 8, 8), pl.ds(0, BB)],
                    sem_s[q]).wait()

        prep(0, 0)

        def step(g, carry):
            for p in range(2):
                k = 2 * g + p

                @pl.when(k + 1 < BLK_W)
                def _():
                    prep(k + 1, 1 - p)

                drain_g(p)

                @pl.when(k >= 2)
                def _():
                    drain_s(p)

                transpose(p)
                store(k, p)
            return carry

        lax.fori_loop(0, BLK_W // 2, step, 0)
        for p in range(2):
            drain_s(p)

    return pl.kernel(
        body,
        out_type=jax.ShapeDtypeStruct((HIST, 8, BATCH // BB, 8, BB),
                                      jnp.float32),
        mesh=mesh,
        compiler_params=pltpu.CompilerParams(use_tc_tiling_on_sc=False,
                                             needs_layout_passes=False),
        scratch_types=[
            pltpu.VMEM((IDS_W,), jnp.int32),
            pltpu.VMEM((2, DIGESTS, BB), jnp.int32),
            pltpu.VMEM((2, DIGESTS * BB, KEY_DIM), jnp.float32),
            pltpu.VMEM((2, DIGESTS * KEY_DIM, 136), jnp.float32),
            pltpu.SemaphoreType.DMA,
            pltpu.SemaphoreType.DMA,
            pltpu.SemaphoreType.DMA,
            pltpu.SemaphoreType.DMA,
        ],
    )


def kernel(x, lut):
    xt = x.reshape(BATCH // BB, BB, HIST // 8, 8)
    xt = xt.transpose(2, 0, 3, 1).reshape(BATCH * HIST)
    out5 = _make_kernel()(xt, lut)
    return out5.transpose(2, 4, 0, 1, 3).reshape(BATCH, HIST,
                                                 DIGESTS * KEY_DIM)
